# baseline (speedup 1.0000x reference)
import functools

import jax
import jax.numpy as jnp
from jax import lax
from jax.experimental import pallas as pl
from jax.experimental.pallas import tpu as pltpu
from jax.experimental.pallas import tpu_sc as plsc

N = 5000
D = 128
E = 160000
NPAD = 5120
BQ = 512
NB = NPAD // BQ

NC = 2
NS = 16
CHUNK = 128
CH = 40
EPT = NC * NS * CH * CHUNK
RPT = NPAD // NS


def _attn_body(x_ref, wi_ref, bi_ref, wo0_ref, bo0_ref, o_ref, k_s, rs_s):
    x = x_ref[...]
    wk = wi_ref[D:2 * D, :]
    bk = bi_ref[0:1, D:2 * D]
    k_s[pl.ds(0, N), :] = lax.dot_general(
        x, wk, (((1,), (1,)), ((), ())),
        preferred_element_type=jnp.float32) + bk
    k_s[pl.ds(N, NPAD - N), :] = jnp.zeros((NPAD - N, D), jnp.float32)
    wv = wi_ref[2 * D:3 * D, :]
    bv = bi_ref[0:1, 2 * D:3 * D]
    wo0 = wo0_ref[...]
    u = lax.dot_general(
        wo0, wv, (((1,), (0,)), ((), ())),
        preferred_element_type=jnp.float32)
    c0 = jnp.sum(bv * wo0)
    wc = jnp.sum(x * u, axis=1, keepdims=True)
    rs_s[pl.ds(0, N), :] = jnp.concatenate(
        [jnp.ones((N, 1), jnp.float32), wc + c0], axis=1).astype(jnp.bfloat16)
    rs_s[pl.ds(N, NPAD - N), :] = jnp.zeros((NPAD - N, 2), jnp.bfloat16)

    wq = wi_ref[0:D, :]
    bq = bi_ref[0:1, 0:D]
    scale = 1.0 / jnp.sqrt(jnp.float32(D))

    def one_block(row0):
        xq = x_ref[pl.ds(row0, BQ), :]
        q = (lax.dot_general(
            xq, wq, (((1,), (1,)), ((), ())),
            preferred_element_type=jnp.float32) + bq) * scale
        logits = lax.dot_general(
            q.astype(jnp.bfloat16), k_s[...].astype(jnp.bfloat16),
            (((1,), (1,)), ((), ())),
            preferred_element_type=jnp.float32)
        p = jnp.exp(logits.astype(jnp.bfloat16))
        res = lax.dot_general(
            p, rs_s[...], (((1,), (0,)), ((), ())),
            preferred_element_type=jnp.float32)
        o_ref[pl.ds(row0, BQ), :] = (res[:, 1:2] / res[:, 0:1]
                                     + bo0_ref[...])

    def qblock(i, carry):
        one_block(i * BQ)
        return carry

    lax.fori_loop(0, NB - 1, qblock, 0)
    one_block(N - BQ)


def _attn_call(x, wi, bi, wo0, bo0, interpret=False):
    return pl.pallas_call(
        _attn_body,
        out_shape=jax.ShapeDtypeStruct((N, 1), jnp.float32),
        scratch_shapes=[
            pltpu.VMEM((NPAD, D), jnp.float32),
            pltpu.VMEM((NPAD, 2), jnp.bfloat16),
        ],
        interpret=interpret,
    )(x, wi, bi, wo0, bo0)


def _sc_body(xs_hbm, src_hbm, dst_hbm, ones_hbm, zeros_hbm, out_hbm,
             src_v, dst_v, vals_v, ones_v, sum_v, cnt_v,
             acc_sh, cnt_sh, xs_sh, sem):
    c = lax.axis_index("c")
    s = lax.axis_index("s")

    @pl.when(s == 0)
    def _():
        pltpu.sync_copy(zeros_hbm, acc_sh)
        pltpu.sync_copy(zeros_hbm, cnt_sh)
        pltpu.sync_copy(xs_hbm, xs_sh)

    pltpu.sync_copy(ones_hbm.at[0], ones_v)
    pltpu.sync_copy(src_hbm.at[c, s], src_v)
    pltpu.sync_copy(dst_hbm.at[c, s], dst_v)
    plsc.subcore_barrier()

    vbuf = (vals_v.at[0], vals_v.at[1])
    sems = (sem.at[0], sem.at[1])
    pltpu.async_copy(xs_sh.at[src_v.at[0]], vbuf[0], sems[0])

    def chunk_body(j2, carry):
        j = j2 * 2
        pltpu.async_copy(xs_sh.at[src_v.at[j + 1]], vbuf[1], sems[1])
        pltpu.make_async_copy(ones_hbm.at[0], vbuf[0], sems[0]).wait()
        pltpu.sync_copy(vbuf[0], acc_sh.at[dst_v.at[j]], add=True)
        pltpu.sync_copy(ones_v, cnt_sh.at[dst_v.at[j]], add=True)

        @pl.when(j2 + 1 < CH // 2)
        def _():
            pltpu.async_copy(xs_sh.at[src_v.at[j + 2]], vbuf[0], sems[0])

        pltpu.make_async_copy(ones_hbm.at[0], vbuf[1], sems[1]).wait()
        pltpu.sync_copy(vbuf[1], acc_sh.at[dst_v.at[j + 1]], add=True)
        pltpu.sync_copy(ones_v, cnt_sh.at[dst_v.at[j + 1]], add=True)
        return carry

    lax.fori_loop(0, CH // 2, chunk_body, 0)
    plsc.subcore_barrier()

    base = s * RPT
    pltpu.sync_copy(acc_sh.at[pl.ds(base, RPT)], sum_v)
    pltpu.sync_copy(cnt_sh.at[pl.ds(base, RPT)], cnt_v)
    pltpu.sync_copy(sum_v, out_hbm.at[pl.ds(2 * c * NPAD + base, RPT)])
    pltpu.sync_copy(cnt_v, out_hbm.at[pl.ds((2 * c + 1) * NPAD + base, RPT)])


@functools.cache
def _sc_segment_sum_fn():
    return pl.kernel(
        _sc_body,
        out_type=jax.ShapeDtypeStruct((4 * NPAD,), jnp.float32),
        mesh=plsc.VectorSubcoreMesh(core_axis_name="c", subcore_axis_name="s",
                                    num_cores=NC, num_subcores=NS),
        scratch_types=[
            pltpu.VMEM((CH, CHUNK), jnp.int32),
            pltpu.VMEM((CH, CHUNK), jnp.int32),
            pltpu.VMEM((2, CHUNK), jnp.float32),
            pltpu.VMEM((CHUNK,), jnp.float32),
            pltpu.VMEM((RPT,), jnp.float32),
            pltpu.VMEM((RPT,), jnp.float32),
            pltpu.VMEM_SHARED((NPAD,), jnp.float32),
            pltpu.VMEM_SHARED((NPAD,), jnp.float32),
            pltpu.VMEM_SHARED((N,), jnp.float32),
            pltpu.SemaphoreType.DMA((2,)),
        ],
    )


def _sc_segment_sum(xs_tab, src_g, dst_g, ones_h, zeros_h):
    return _sc_segment_sum_fn()(xs_tab, src_g, dst_g, ones_h, zeros_h)


def _expand_body(pa_ref, pb_ref, xu_ref, xi_ref, wl_ref, bl_ref, wr_ref,
                 o0_ref, o1_ref):
    sb = pb_ref[0] + pb_ref[2]
    cb = pb_ref[1] + pb_ref[3]
    mean_b = jnp.where(cb > 0.0, sb / jnp.maximum(cb, 1.0), 0.0)
    o0_ref[...] = mean_b * wl_ref[0] + bl_ref[0] + xu_ref[...] * wr_ref[0]
    sa = pa_ref[0] + pa_ref[2]
    ca = pa_ref[1] + pa_ref[3]
    mean_a = jnp.where(ca > 0.0, sa / jnp.maximum(ca, 1.0), 0.0)
    o1_ref[...] = mean_a * wl_ref[1] + bl_ref[1] + xi_ref[...] * wr_ref[1]


def _expand_call(pa, pb, xs_u, xs_i, wl_all, bl_all, wr_all,
                 interpret=False):
    be = 512
    par = pl.BlockSpec((4, be, 1), lambda i: (0, i, 0))
    vec = pl.BlockSpec((be, 1), lambda i: (i, 0))
    wrow = pl.BlockSpec((2, 1, D), lambda i: (0, 0, 0))
    return pl.pallas_call(
        _expand_body,
        grid=(NPAD // be,),
        in_specs=[par, par, vec, vec, wrow, wrow, wrow],
        out_specs=[
            pl.BlockSpec((be, D), lambda i: (i, 0)),
            pl.BlockSpec((be, D), lambda i: (i, 0)),
        ],
        out_shape=[
            jax.ShapeDtypeStruct((N, D), jnp.float32),
            jax.ShapeDtypeStruct((N, D), jnp.float32),
        ],
        interpret=interpret,
    )(pa, pb, xs_u, xs_i, wl_all, bl_all, wr_all)


def _prep_edges(edge_index):
    pad_n = EPT - E
    ar = jnp.arange(pad_n, dtype=jnp.int32)
    src = jnp.concatenate([edge_index[0], ar % N])
    dst = jnp.concatenate([edge_index[1], N + (ar % (NPAD - N))])
    return (src.reshape(NC, NS, CH, CHUNK), dst.reshape(NC, NS, CH, CHUNK))


def kernel(x_user, x_item, edge_index_ui, edge_index_iu,
           Wi_user, bi_user, Wo_user, bo_user,
           Wi_item, bi_item, Wo_item, bo_item,
           Wl_ui, bl_ui, Wr_ui, Wl_iu, bl_iu, Wr_iu):
    ones_h = jnp.ones((1, CHUNK), jnp.float32)
    zeros_h = jnp.zeros((NPAD,), jnp.float32)
    src_a, dst_a = _prep_edges(edge_index_ui)
    src_b, dst_b = _prep_edges(edge_index_iu)

    xs_u = _attn_call(x_user, Wi_user, bi_user.reshape(1, 3 * D),
                      Wo_user[0:1, :], bo_user[0:1].reshape(1, 1))
    pa = _sc_segment_sum(xs_u[:, 0], src_a, dst_a, ones_h, zeros_h)
    xs_i = _attn_call(x_item, Wi_item, bi_item.reshape(1, 3 * D),
                      Wo_item[0:1, :], bo_item[0:1].reshape(1, 1))
    pb = _sc_segment_sum(xs_i[:, 0], src_b, dst_b, ones_h, zeros_h)

    wl_all = jnp.stack([Wl_iu[:, 0], Wl_ui[:, 0]]).reshape(2, 1, D)
    bl_all = jnp.stack([bl_iu, bl_ui]).reshape(2, 1, D)
    wr_all = jnp.stack([Wr_iu[:, 0], Wr_ui[:, 0]]).reshape(2, 1, D)
    o0, o1 = _expand_call(pa.reshape(4, NPAD, 1), pb.reshape(4, NPAD, 1),
                          xs_u, xs_i, wl_all, bl_all, wr_all)
    return (o0, o1)


# speedup vs baseline: 1.0545x; 1.0545x over previous
import functools

import jax
import jax.numpy as jnp
from jax import lax
from jax.experimental import pallas as pl
from jax.experimental.pallas import tpu as pltpu
from jax.experimental.pallas import tpu_sc as plsc

N = 5000
D = 128
E = 160000
NPAD = 5120
BQ = 1024
NB = NPAD // BQ
NEG = -1e30

NC = 2
NS = 16
CHUNK = 128
CH = 80
EPT = NS * CH * CHUNK
RPT = NPAD // NS


def _attn_one(x_ref, wi_ref, bi_ref, wo0_ref, bo0_ref, o_ref, k_s, rs_s):
    x = x_ref[...]
    wk = wi_ref[0, D:2 * D, :]
    bk = bi_ref[0, 0:1, D:2 * D]
    k_s[pl.ds(0, N), :] = lax.dot_general(
        x, wk, (((1,), (1,)), ((), ())),
        preferred_element_type=jnp.float32) + bk
    k_s[pl.ds(N, NPAD - N), :] = jnp.zeros((NPAD - N, D), jnp.float32)
    wv = wi_ref[0, 2 * D:3 * D, :]
    bv = bi_ref[0, 0:1, 2 * D:3 * D]
    wo0 = wo0_ref[0]
    u = lax.dot_general(
        wo0, wv, (((1,), (0,)), ((), ())),
        preferred_element_type=jnp.float32)
    c0 = jnp.sum(bv * wo0)
    wc = jnp.sum(x * u, axis=1, keepdims=True)
    rs_s[pl.ds(0, N), :] = jnp.concatenate(
        [jnp.ones((N, 1), jnp.float32), wc + c0], axis=1).astype(jnp.bfloat16)
    rs_s[pl.ds(N, NPAD - N), :] = jnp.zeros((NPAD - N, 2), jnp.bfloat16)

    wq = wi_ref[0, 0:D, :]
    bq = bi_ref[0, 0:1, 0:D]
    scale = 1.0 / jnp.sqrt(jnp.float32(D))

    def qblock(i, carry):
        xq = x_ref[pl.ds(i * BQ, BQ), :]
        q = (lax.dot_general(
            xq, wq, (((1,), (1,)), ((), ())),
            preferred_element_type=jnp.float32) + bq) * scale
        logits = lax.dot_general(
            q.astype(jnp.bfloat16), k_s[...].astype(jnp.bfloat16),
            (((1,), (1,)), ((), ())),
            preferred_element_type=jnp.float32)
        p = jnp.exp(logits.astype(jnp.bfloat16))
        res = lax.dot_general(
            p, rs_s[...], (((1,), (0,)), ((), ())),
            preferred_element_type=jnp.float32)
        o_ref[pl.ds(i * BQ, BQ), :] = (res[:, 1:2] / res[:, 0:1]
                                       + bo0_ref[0])
        return carry

    lax.fori_loop(0, NB - 1, qblock, 0)
    i = NB - 1
    xq = x_ref[pl.ds(N - BQ, BQ), :]
    q = (lax.dot_general(
        xq, wq, (((1,), (1,)), ((), ())),
        preferred_element_type=jnp.float32) + bq) * scale
    logits = lax.dot_general(
        q.astype(jnp.bfloat16), k_s[...].astype(jnp.bfloat16),
        (((1,), (1,)), ((), ())),
        preferred_element_type=jnp.float32)
    p = jnp.exp(logits.astype(jnp.bfloat16))
    res = lax.dot_general(
        p, rs_s[...], (((1,), (0,)), ((), ())),
        preferred_element_type=jnp.float32)
    o_ref[pl.ds(N - BQ, BQ), :] = res[:, 1:2] / res[:, 0:1] + bo0_ref[0]


def _attn_body(xu_ref, xi_ref, wi_ref, bi_ref, wo0_ref, bo0_ref,
               ou_ref, oi_ref, k_s, rs_s):
    _attn_one(xu_ref, wi_ref.at[0:1], bi_ref.at[0:1], wo0_ref.at[0:1],
              bo0_ref.at[0:1], ou_ref, k_s, rs_s)
    _attn_one(xi_ref, wi_ref.at[1:2], bi_ref.at[1:2], wo0_ref.at[1:2],
              bo0_ref.at[1:2], oi_ref, k_s, rs_s)


def _attn_call(x_user, x_item, wi_all, bi_all, wo0_all, bo0_all,
               interpret=False):
    return pl.pallas_call(
        _attn_body,
        out_shape=[
            jax.ShapeDtypeStruct((N, 1), jnp.float32),
            jax.ShapeDtypeStruct((N, 1), jnp.float32),
        ],
        scratch_shapes=[
            pltpu.VMEM((NPAD, D), jnp.float32),
            pltpu.VMEM((NPAD, 2), jnp.bfloat16),
        ],
        interpret=interpret,
    )(x_user, x_item, wi_all, bi_all, wo0_all, bo0_all)


def _sc_body(xs_hbm, src_hbm, dst_hbm, ones_hbm, zeros_hbm, out_hbm,
             src_v, dst_v, vals_v, ones_v, sum_v, cnt_v, mean_v,
             acc_sh, cnt_sh, xs_sh, sem):
    c = lax.axis_index("c")
    s = lax.axis_index("s")

    @pl.when(s == 0)
    def _():
        pltpu.sync_copy(zeros_hbm, acc_sh)
        pltpu.sync_copy(zeros_hbm, cnt_sh)
        pltpu.sync_copy(xs_hbm.at[pl.ds((1 - c) * NPAD, NPAD)], xs_sh)

    pltpu.sync_copy(ones_hbm.at[0], ones_v)
    pltpu.sync_copy(src_hbm.at[c, s], src_v)
    pltpu.sync_copy(dst_hbm.at[c, s], dst_v)
    plsc.subcore_barrier()

    vbuf = (vals_v.at[0], vals_v.at[1])
    sems = (sem.at[0], sem.at[1])
    pltpu.async_copy(xs_sh.at[src_v.at[0]], vbuf[0], sems[0])

    def chunk_body(j2, carry):
        j = j2 * 2
        pltpu.async_copy(xs_sh.at[src_v.at[j + 1]], vbuf[1], sems[1])
        pltpu.make_async_copy(ones_hbm.at[0], vbuf[0], sems[0]).wait()
        pltpu.sync_copy(vbuf[0], acc_sh.at[dst_v.at[j]], add=True)
        pltpu.sync_copy(ones_v, cnt_sh.at[dst_v.at[j]], add=True)

        @pl.when(j2 + 1 < CH // 2)
        def _():
            pltpu.async_copy(xs_sh.at[src_v.at[j + 2]], vbuf[0], sems[0])

        pltpu.make_async_copy(ones_hbm.at[0], vbuf[1], sems[1]).wait()
        pltpu.sync_copy(vbuf[1], acc_sh.at[dst_v.at[j + 1]], add=True)
        pltpu.sync_copy(ones_v, cnt_sh.at[dst_v.at[j + 1]], add=True)
        return carry

    lax.fori_loop(0, CH // 2, chunk_body, 0)
    plsc.subcore_barrier()

    base = s * RPT
    pltpu.sync_copy(acc_sh.at[pl.ds(base, RPT)], sum_v)
    pltpu.sync_copy(cnt_sh.at[pl.ds(base, RPT)], cnt_v)

    def fin(i, carry):
        sl = pl.ds(i * 16, 16)
        sv = sum_v[sl]
        cv = cnt_v[sl]
        mean_v[sl] = jnp.where(cv > 0.0, sv / jnp.maximum(cv, 1.0), 0.0)
        return carry

    lax.fori_loop(0, RPT // 16, fin, 0)
    pltpu.sync_copy(mean_v, out_hbm.at[pl.ds(c * NPAD + base, RPT)])


@functools.cache
def _sc_segment_mean_fn():
    return pl.kernel(
        _sc_body,
        out_type=jax.ShapeDtypeStruct((2 * NPAD,), jnp.float32),
        mesh=plsc.VectorSubcoreMesh(core_axis_name="c", subcore_axis_name="s",
                                    num_cores=NC, num_subcores=NS),
        scratch_types=[
            pltpu.VMEM((CH, CHUNK), jnp.int32),
            pltpu.VMEM((CH, CHUNK), jnp.int32),
            pltpu.VMEM((2, CHUNK), jnp.float32),
            pltpu.VMEM((CHUNK,), jnp.float32),
            pltpu.VMEM((RPT,), jnp.float32),
            pltpu.VMEM((RPT,), jnp.float32),
            pltpu.VMEM((RPT,), jnp.float32),
            pltpu.VMEM_SHARED((NPAD,), jnp.float32),
            pltpu.VMEM_SHARED((NPAD,), jnp.float32),
            pltpu.VMEM_SHARED((NPAD,), jnp.float32),
            pltpu.SemaphoreType.DMA((2,)),
        ],
    )


def _sc_segment_mean(xs_flat, src_g, dst_g, ones_h, zeros_h):
    return _sc_segment_mean_fn()(xs_flat, src_g, dst_g, ones_h, zeros_h)


def _expand_body(m0_ref, m1_ref, x0_ref, x1_ref, wl_ref, bl_ref, wr_ref,
                 o0_ref, o1_ref):
    o0_ref[...] = m0_ref[...] * wl_ref[0] + bl_ref[0] + x0_ref[...] * wr_ref[0]
    o1_ref[...] = m1_ref[...] * wl_ref[1] + bl_ref[1] + x1_ref[...] * wr_ref[1]


def _expand_call(mean0, mean1, xs0, xs1, wl_all, bl_all, wr_all,
                 interpret=False):
    be = 512
    vec = pl.BlockSpec((be, 1), lambda i: (i, 0))
    wrow = pl.BlockSpec((2, 1, D), lambda i: (0, 0, 0))
    return pl.pallas_call(
        _expand_body,
        grid=(NPAD // be,),
        in_specs=[vec, vec, vec, vec, wrow, wrow, wrow],
        out_specs=[
            pl.BlockSpec((be, D), lambda i: (i, 0)),
            pl.BlockSpec((be, D), lambda i: (i, 0)),
        ],
        out_shape=[
            jax.ShapeDtypeStruct((N, D), jnp.float32),
            jax.ShapeDtypeStruct((N, D), jnp.float32),
        ],
        interpret=interpret,
    )(mean0, mean1, xs0, xs1, wl_all, bl_all, wr_all)


def _prep_edges(edge_index_iu, edge_index_ui):
    pad_n = EPT - E
    ar = jnp.arange(pad_n, dtype=jnp.int32)
    pad_src = ar % N
    pad_dst = N + (ar % (NPAD - N))
    src0 = jnp.concatenate([edge_index_iu[0], pad_src])
    dst0 = jnp.concatenate([edge_index_iu[1], pad_dst])
    src1 = jnp.concatenate([edge_index_ui[0], pad_src])
    dst1 = jnp.concatenate([edge_index_ui[1], pad_dst])
    src_g = jnp.stack([src0, src1]).reshape(2, NS, CH, CHUNK)
    dst_g = jnp.stack([dst0, dst1]).reshape(2, NS, CH, CHUNK)
    return src_g, dst_g


def kernel(x_user, x_item, edge_index_ui, edge_index_iu,
           Wi_user, bi_user, Wo_user, bo_user,
           Wi_item, bi_item, Wo_item, bo_item,
           Wl_ui, bl_ui, Wr_ui, Wl_iu, bl_iu, Wr_iu):
    wi_all = jnp.stack([Wi_user, Wi_item])
    bi_all = jnp.stack([bi_user, bi_item]).reshape(2, 1, 3 * D)
    wo0_all = jnp.stack([Wo_user[0:1, :], Wo_item[0:1, :]])
    bo0_all = jnp.stack([bo_user[0:1], bo_item[0:1]]).reshape(2, 1, 1)

    xs_u, xs_i = _attn_call(x_user, x_item, wi_all, bi_all, wo0_all, bo0_all)

    src_g, dst_g = _prep_edges(edge_index_iu, edge_index_ui)
    pad1 = jnp.zeros((NPAD - N,), jnp.float32)
    xs_flat = jnp.concatenate(
        [xs_u[:, 0], pad1, xs_i[:, 0], pad1])
    ones_h = jnp.ones((1, CHUNK), jnp.float32)
    zeros_h = jnp.zeros((NPAD,), jnp.float32)
    mean_all = _sc_segment_mean(xs_flat, src_g, dst_g, ones_h, zeros_h)

    wl_all = jnp.stack([Wl_iu[:, 0], Wl_ui[:, 0]]).reshape(2, 1, D)
    bl_all = jnp.stack([bl_iu, bl_ui]).reshape(2, 1, D)
    wr_all = jnp.stack([Wr_iu[:, 0], Wr_ui[:, 0]]).reshape(2, 1, D)
    mean2 = mean_all.reshape(2, NPAD, 1)
    o0, o1 = _expand_call(mean2[0], mean2[1], xs_u, xs_i,
                          wl_all, bl_all, wr_all)
    return (o0, o1)


# speedup vs baseline: 1.0587x; 1.0040x over previous
import functools

import jax
import jax.numpy as jnp
from jax import lax
from jax.experimental import pallas as pl
from jax.experimental.pallas import tpu as pltpu
from jax.experimental.pallas import tpu_sc as plsc

N = 5000
D = 128
E = 160000
NPAD = 5120
BQ = 1280
NB = NPAD // BQ
NEG = -1e30

NC = 2
NS = 16
CHUNK = 128
CH = 80
EPT = NS * CH * CHUNK
RPT = NPAD // NS


def _attn_one(x_ref, wi_ref, bi_ref, wo0_ref, bo0_ref, o_ref, k_s, rs_s):
    x = x_ref[...]
    wk = wi_ref[0, D:2 * D, :]
    bk = bi_ref[0, 0:1, D:2 * D]
    k_s[pl.ds(0, N), :] = lax.dot_general(
        x, wk, (((1,), (1,)), ((), ())),
        preferred_element_type=jnp.float32) + bk
    k_s[pl.ds(N, NPAD - N), :] = jnp.zeros((NPAD - N, D), jnp.float32)
    wv = wi_ref[0, 2 * D:3 * D, :]
    bv = bi_ref[0, 0:1, 2 * D:3 * D]
    wo0 = wo0_ref[0]
    u = lax.dot_general(
        wo0, wv, (((1,), (0,)), ((), ())),
        preferred_element_type=jnp.float32)
    c0 = jnp.sum(bv * wo0)
    wc = jnp.sum(x * u, axis=1, keepdims=True)
    rs_s[pl.ds(0, N), :] = jnp.concatenate(
        [jnp.ones((N, 1), jnp.float32), wc + c0], axis=1).astype(jnp.bfloat16)
    rs_s[pl.ds(N, NPAD - N), :] = jnp.zeros((NPAD - N, 2), jnp.bfloat16)

    wq = wi_ref[0, 0:D, :]
    bq = bi_ref[0, 0:1, 0:D]
    scale = 1.0 / jnp.sqrt(jnp.float32(D))

    def qblock(i, carry):
        xq = x_ref[pl.ds(i * BQ, BQ), :]
        q = (lax.dot_general(
            xq, wq, (((1,), (1,)), ((), ())),
            preferred_element_type=jnp.float32) + bq) * scale
        logits = lax.dot_general(
            q.astype(jnp.bfloat16), k_s[...].astype(jnp.bfloat16),
            (((1,), (1,)), ((), ())),
            preferred_element_type=jnp.float32)
        p = jnp.exp(logits.astype(jnp.bfloat16))
        res = lax.dot_general(
            p, rs_s[...], (((1,), (0,)), ((), ())),
            preferred_element_type=jnp.float32)
        o_ref[pl.ds(i * BQ, BQ), :] = (res[:, 1:2] / res[:, 0:1]
                                       + bo0_ref[0])
        return carry

    lax.fori_loop(0, NB - 1, qblock, 0)
    i = NB - 1
    xq = x_ref[pl.ds(N - BQ, BQ), :]
    q = (lax.dot_general(
        xq, wq, (((1,), (1,)), ((), ())),
        preferred_element_type=jnp.float32) + bq) * scale
    logits = lax.dot_general(
        q.astype(jnp.bfloat16), k_s[...].astype(jnp.bfloat16),
        (((1,), (1,)), ((), ())),
        preferred_element_type=jnp.float32)
    p = jnp.exp(logits.astype(jnp.bfloat16))
    res = lax.dot_general(
        p, rs_s[...], (((1,), (0,)), ((), ())),
        preferred_element_type=jnp.float32)
    o_ref[pl.ds(N - BQ, BQ), :] = res[:, 1:2] / res[:, 0:1] + bo0_ref[0]


def _attn_body(xu_ref, xi_ref, wi_ref, bi_ref, wo0_ref, bo0_ref,
               ou_ref, oi_ref, k_s, rs_s):
    _attn_one(xu_ref, wi_ref.at[0:1], bi_ref.at[0:1], wo0_ref.at[0:1],
              bo0_ref.at[0:1], ou_ref, k_s, rs_s)
    _attn_one(xi_ref, wi_ref.at[1:2], bi_ref.at[1:2], wo0_ref.at[1:2],
              bo0_ref.at[1:2], oi_ref, k_s, rs_s)


def _attn_call(x_user, x_item, wi_all, bi_all, wo0_all, bo0_all,
               interpret=False):
    return pl.pallas_call(
        _attn_body,
        out_shape=[
            jax.ShapeDtypeStruct((N, 1), jnp.float32),
            jax.ShapeDtypeStruct((N, 1), jnp.float32),
        ],
        scratch_shapes=[
            pltpu.VMEM((NPAD, D), jnp.float32),
            pltpu.VMEM((NPAD, 2), jnp.bfloat16),
        ],
        interpret=interpret,
    )(x_user, x_item, wi_all, bi_all, wo0_all, bo0_all)


def _sc_body(xs_hbm, src_hbm, dst_hbm, ones_hbm, zeros_hbm, out_hbm,
             src_v, dst_v, vals_v, ones_v, sum_v, cnt_v, mean_v,
             acc_sh, cnt_sh, xs_sh, sem):
    c = lax.axis_index("c")
    s = lax.axis_index("s")

    @pl.when(s == 0)
    def _():
        pltpu.sync_copy(zeros_hbm, acc_sh)
        pltpu.sync_copy(zeros_hbm, cnt_sh)
        pltpu.sync_copy(xs_hbm.at[pl.ds((1 - c) * NPAD, NPAD)], xs_sh)

    pltpu.sync_copy(ones_hbm.at[0], ones_v)
    pltpu.sync_copy(src_hbm.at[c, s], src_v)
    pltpu.sync_copy(dst_hbm.at[c, s], dst_v)
    plsc.subcore_barrier()

    vbuf = (vals_v.at[0], vals_v.at[1])
    sems = (sem.at[0], sem.at[1])
    pltpu.async_copy(xs_sh.at[src_v.at[0]], vbuf[0], sems[0])

    def chunk_body(j2, carry):
        j = j2 * 2
        pltpu.async_copy(xs_sh.at[src_v.at[j + 1]], vbuf[1], sems[1])
        pltpu.make_async_copy(ones_hbm.at[0], vbuf[0], sems[0]).wait()
        pltpu.sync_copy(vbuf[0], acc_sh.at[dst_v.at[j]], add=True)
        pltpu.sync_copy(ones_v, cnt_sh.at[dst_v.at[j]], add=True)

        @pl.when(j2 + 1 < CH // 2)
        def _():
            pltpu.async_copy(xs_sh.at[src_v.at[j + 2]], vbuf[0], sems[0])

        pltpu.make_async_copy(ones_hbm.at[0], vbuf[1], sems[1]).wait()
        pltpu.sync_copy(vbuf[1], acc_sh.at[dst_v.at[j + 1]], add=True)
        pltpu.sync_copy(ones_v, cnt_sh.at[dst_v.at[j + 1]], add=True)
        return carry

    lax.fori_loop(0, CH // 2, chunk_body, 0)
    plsc.subcore_barrier()

    base = s * RPT
    pltpu.sync_copy(acc_sh.at[pl.ds(base, RPT)], sum_v)
    pltpu.sync_copy(cnt_sh.at[pl.ds(base, RPT)], cnt_v)

    def fin(i, carry):
        sl = pl.ds(i * 16, 16)
        sv = sum_v[sl]
        cv = cnt_v[sl]
        mean_v[sl] = jnp.where(cv > 0.0, sv / jnp.maximum(cv, 1.0), 0.0)
        return carry

    lax.fori_loop(0, RPT // 16, fin, 0)
    pltpu.sync_copy(mean_v, out_hbm.at[pl.ds(c * NPAD + base, RPT)])


@functools.cache
def _sc_segment_mean_fn():
    return pl.kernel(
        _sc_body,
        out_type=jax.ShapeDtypeStruct((2 * NPAD,), jnp.float32),
        mesh=plsc.VectorSubcoreMesh(core_axis_name="c", subcore_axis_name="s",
                                    num_cores=NC, num_subcores=NS),
        scratch_types=[
            pltpu.VMEM((CH, CHUNK), jnp.int32),
            pltpu.VMEM((CH, CHUNK), jnp.int32),
            pltpu.VMEM((2, CHUNK), jnp.float32),
            pltpu.VMEM((CHUNK,), jnp.float32),
            pltpu.VMEM((RPT,), jnp.float32),
            pltpu.VMEM((RPT,), jnp.float32),
            pltpu.VMEM((RPT,), jnp.float32),
            pltpu.VMEM_SHARED((NPAD,), jnp.float32),
            pltpu.VMEM_SHARED((NPAD,), jnp.float32),
            pltpu.VMEM_SHARED((NPAD,), jnp.float32),
            pltpu.SemaphoreType.DMA((2,)),
        ],
    )


def _sc_segment_mean(xs_flat, src_g, dst_g, ones_h, zeros_h):
    return _sc_segment_mean_fn()(xs_flat, src_g, dst_g, ones_h, zeros_h)


def _expand_body(m0_ref, m1_ref, x0_ref, x1_ref, wl_ref, bl_ref, wr_ref,
                 o0_ref, o1_ref):
    o0_ref[...] = m0_ref[...] * wl_ref[0] + bl_ref[0] + x0_ref[...] * wr_ref[0]
    o1_ref[...] = m1_ref[...] * wl_ref[1] + bl_ref[1] + x1_ref[...] * wr_ref[1]


def _expand_call(mean0, mean1, xs0, xs1, wl_all, bl_all, wr_all,
                 interpret=False):
    be = 512
    vec = pl.BlockSpec((be, 1), lambda i: (i, 0))
    wrow = pl.BlockSpec((2, 1, D), lambda i: (0, 0, 0))
    return pl.pallas_call(
        _expand_body,
        grid=(NPAD // be,),
        in_specs=[vec, vec, vec, vec, wrow, wrow, wrow],
        out_specs=[
            pl.BlockSpec((be, D), lambda i: (i, 0)),
            pl.BlockSpec((be, D), lambda i: (i, 0)),
        ],
        out_shape=[
            jax.ShapeDtypeStruct((N, D), jnp.float32),
            jax.ShapeDtypeStruct((N, D), jnp.float32),
        ],
        interpret=interpret,
    )(mean0, mean1, xs0, xs1, wl_all, bl_all, wr_all)


def _prep_edges(edge_index_iu, edge_index_ui):
    pad_n = EPT - E
    ar = jnp.arange(pad_n, dtype=jnp.int32)
    pad_src = ar % N
    pad_dst = N + (ar % (NPAD - N))
    src0 = jnp.concatenate([edge_index_iu[0], pad_src])
    dst0 = jnp.concatenate([edge_index_iu[1], pad_dst])
    src1 = jnp.concatenate([edge_index_ui[0], pad_src])
    dst1 = jnp.concatenate([edge_index_ui[1], pad_dst])
    src_g = jnp.stack([src0, src1]).reshape(2, NS, CH, CHUNK)
    dst_g = jnp.stack([dst0, dst1]).reshape(2, NS, CH, CHUNK)
    return src_g, dst_g


def kernel(x_user, x_item, edge_index_ui, edge_index_iu,
           Wi_user, bi_user, Wo_user, bo_user,
           Wi_item, bi_item, Wo_item, bo_item,
           Wl_ui, bl_ui, Wr_ui, Wl_iu, bl_iu, Wr_iu):
    wi_all = jnp.stack([Wi_user, Wi_item])
    bi_all = jnp.stack([bi_user, bi_item]).reshape(2, 1, 3 * D)
    wo0_all = jnp.stack([Wo_user[0:1, :], Wo_item[0:1, :]])
    bo0_all = jnp.stack([bo_user[0:1], bo_item[0:1]]).reshape(2, 1, 1)

    xs_u, xs_i = _attn_call(x_user, x_item, wi_all, bi_all, wo0_all, bo0_all)

    src_g, dst_g = _prep_edges(edge_index_iu, edge_index_ui)
    pad1 = jnp.zeros((NPAD - N,), jnp.float32)
    xs_flat = jnp.concatenate(
        [xs_u[:, 0], pad1, xs_i[:, 0], pad1])
    ones_h = jnp.ones((1, CHUNK), jnp.float32)
    zeros_h = jnp.zeros((NPAD,), jnp.float32)
    mean_all = _sc_segment_mean(xs_flat, src_g, dst_g, ones_h, zeros_h)

    wl_all = jnp.stack([Wl_iu[:, 0], Wl_ui[:, 0]]).reshape(2, 1, D)
    bl_all = jnp.stack([bl_iu, bl_ui]).reshape(2, 1, D)
    wr_all = jnp.stack([Wr_iu[:, 0], Wr_ui[:, 0]]).reshape(2, 1, D)
    mean2 = mean_all.reshape(2, NPAD, 1)
    o0, o1 = _expand_call(mean2[0], mean2[1], xs_u, xs_i,
                          wl_all, bl_all, wr_all)
    return (o0, o1)


# speedup vs baseline: 1.0981x; 1.0372x over previous
import functools

import jax
import jax.numpy as jnp
from jax import lax
from jax.experimental import pallas as pl
from jax.experimental.pallas import tpu as pltpu
from jax.experimental.pallas import tpu_sc as plsc

N = 5000
D = 128
E = 160000
NPAD = 5120
BQ = 1280
NB = NPAD // BQ

NC = 2
NS = 16
CHUNK = 128
CH = 80
EPT = NS * CH * CHUNK
RPT = NPAD // NS


def _attn_one(x_ref, wi_ref, bi_ref, wo0_ref, bo0_ref, o_ref, k_s, rs_s):
    x = x_ref[...]
    wk = wi_ref[0, D:2 * D, :]
    bk = bi_ref[0, 0:1, D:2 * D]
    k_s[pl.ds(0, N), :] = lax.dot_general(
        x, wk, (((1,), (1,)), ((), ())),
        preferred_element_type=jnp.float32) + bk
    k_s[pl.ds(N, NPAD - N), :] = jnp.zeros((NPAD - N, D), jnp.float32)
    wv = wi_ref[0, 2 * D:3 * D, :]
    bv = bi_ref[0, 0:1, 2 * D:3 * D]
    wo0 = wo0_ref[0]
    u = lax.dot_general(
        wo0, wv, (((1,), (0,)), ((), ())),
        preferred_element_type=jnp.float32)
    c0 = jnp.sum(bv * wo0)
    wc = jnp.sum(x * u, axis=1, keepdims=True)
    rs_s[pl.ds(0, N), :] = jnp.concatenate(
        [jnp.ones((N, 1), jnp.float32), wc + c0], axis=1).astype(jnp.bfloat16)
    rs_s[pl.ds(N, NPAD - N), :] = jnp.zeros((NPAD - N, 2), jnp.bfloat16)

    wq = wi_ref[0, 0:D, :]
    bq = bi_ref[0, 0:1, 0:D]
    scale = 1.0 / jnp.sqrt(jnp.float32(D))

    def qblock(i, carry):
        xq = x_ref[pl.ds(i * BQ, BQ), :]
        q = (lax.dot_general(
            xq, wq, (((1,), (1,)), ((), ())),
            preferred_element_type=jnp.float32) + bq) * scale
        logits = lax.dot_general(
            q.astype(jnp.bfloat16), k_s[...].astype(jnp.bfloat16),
            (((1,), (1,)), ((), ())),
            preferred_element_type=jnp.float32)
        p = jnp.exp(logits.astype(jnp.bfloat16))
        res = lax.dot_general(
            p, rs_s[...], (((1,), (0,)), ((), ())),
            preferred_element_type=jnp.float32)
        o_ref[pl.ds(i * BQ, BQ), :] = (res[:, 1:2] / res[:, 0:1]
                                       + bo0_ref[0])
        return carry

    lax.fori_loop(0, NB - 1, qblock, 0)
    i = NB - 1
    xq = x_ref[pl.ds(N - BQ, BQ), :]
    q = (lax.dot_general(
        xq, wq, (((1,), (1,)), ((), ())),
        preferred_element_type=jnp.float32) + bq) * scale
    logits = lax.dot_general(
        q.astype(jnp.bfloat16), k_s[...].astype(jnp.bfloat16),
        (((1,), (1,)), ((), ())),
        preferred_element_type=jnp.float32)
    p = jnp.exp(logits.astype(jnp.bfloat16))
    res = lax.dot_general(
        p, rs_s[...], (((1,), (0,)), ((), ())),
        preferred_element_type=jnp.float32)
    o_ref[pl.ds(N - BQ, BQ), :] = res[:, 1:2] / res[:, 0:1] + bo0_ref[0]


def _attn_body(xu_ref, xi_ref, wi_ref, bi_ref, wo0_ref, bo0_ref,
               ou_ref, oi_ref, k_s, rs_s):
    _attn_one(xu_ref, wi_ref.at[0:1], bi_ref.at[0:1], wo0_ref.at[0:1],
              bo0_ref.at[0:1], ou_ref, k_s, rs_s)
    _attn_one(xi_ref, wi_ref.at[1:2], bi_ref.at[1:2], wo0_ref.at[1:2],
              bo0_ref.at[1:2], oi_ref, k_s, rs_s)


def _attn_call(x_user, x_item, wi_all, bi_all, wo0_all, bo0_all,
               interpret=False):
    return pl.pallas_call(
        _attn_body,
        out_shape=[
            jax.ShapeDtypeStruct((N, 1), jnp.float32),
            jax.ShapeDtypeStruct((N, 1), jnp.float32),
        ],
        scratch_shapes=[
            pltpu.VMEM((NPAD, D), jnp.float32),
            pltpu.VMEM((NPAD, 2), jnp.bfloat16),
        ],
        interpret=interpret,
    )(x_user, x_item, wi_all, bi_all, wo0_all, bo0_all)


def _sc_body(xs_hbm, src_hbm, dst_hbm, ones_hbm, zeros_hbm, out_hbm,
             src_v, dst_v, vals_v, ones_v, sum_v, cnt_v, mean_v,
             acc_sh, cnt_sh, xs_sh, sem):
    c = lax.axis_index("c")
    s = lax.axis_index("s")

    @pl.when(s == 0)
    def _():
        pltpu.sync_copy(zeros_hbm, acc_sh)
        pltpu.sync_copy(zeros_hbm, cnt_sh)
        pltpu.sync_copy(xs_hbm.at[pl.ds((1 - c) * NPAD, NPAD)], xs_sh)

    pltpu.sync_copy(ones_hbm.at[0], ones_v)
    pltpu.sync_copy(src_hbm.at[c, s], src_v)
    pltpu.sync_copy(dst_hbm.at[c, s], dst_v)
    plsc.subcore_barrier()

    vbuf = (vals_v.at[0], vals_v.at[1])
    sems = (sem.at[0], sem.at[1])
    pltpu.async_copy(xs_sh.at[src_v.at[0]], vbuf[0], sems[0])

    def chunk_body(j2, carry):
        j = j2 * 2
        pltpu.async_copy(xs_sh.at[src_v.at[j + 1]], vbuf[1], sems[1])
        pltpu.make_async_copy(ones_hbm.at[0], vbuf[0], sems[0]).wait()
        pltpu.sync_copy(vbuf[0], acc_sh.at[dst_v.at[j]], add=True)
        pltpu.sync_copy(ones_v, cnt_sh.at[dst_v.at[j]], add=True)

        @pl.when(j2 + 1 < CH // 2)
        def _():
            pltpu.async_copy(xs_sh.at[src_v.at[j + 2]], vbuf[0], sems[0])

        pltpu.make_async_copy(ones_hbm.at[0], vbuf[1], sems[1]).wait()
        pltpu.sync_copy(vbuf[1], acc_sh.at[dst_v.at[j + 1]], add=True)
        pltpu.sync_copy(ones_v, cnt_sh.at[dst_v.at[j + 1]], add=True)
        return carry

    lax.fori_loop(0, CH // 2, chunk_body, 0)
    plsc.subcore_barrier()

    base = s * RPT
    pltpu.sync_copy(acc_sh.at[pl.ds(base, RPT)], sum_v)
    pltpu.sync_copy(cnt_sh.at[pl.ds(base, RPT)], cnt_v)

    def fin(i, carry):
        sl = pl.ds(i * 16, 16)
        sv = sum_v[sl]
        cv = cnt_v[sl]
        mean_v[sl] = jnp.where(cv > 0.0, sv / jnp.maximum(cv, 1.0), 0.0)
        return carry

    lax.fori_loop(0, RPT // 16, fin, 0)
    pltpu.sync_copy(mean_v, out_hbm.at[pl.ds(c * NPAD + base, RPT)])


@functools.cache
def _sc_segment_mean_fn():
    return pl.kernel(
        _sc_body,
        out_type=jax.ShapeDtypeStruct((2 * NPAD,), jnp.float32),
        mesh=plsc.VectorSubcoreMesh(core_axis_name="c", subcore_axis_name="s",
                                    num_cores=NC, num_subcores=NS),
        scratch_types=[
            pltpu.VMEM((CH, CHUNK), jnp.int32),
            pltpu.VMEM((CH, CHUNK), jnp.int32),
            pltpu.VMEM((2, CHUNK), jnp.float32),
            pltpu.VMEM((CHUNK,), jnp.float32),
            pltpu.VMEM((RPT,), jnp.float32),
            pltpu.VMEM((RPT,), jnp.float32),
            pltpu.VMEM((RPT,), jnp.float32),
            pltpu.VMEM_SHARED((NPAD,), jnp.float32),
            pltpu.VMEM_SHARED((NPAD,), jnp.float32),
            pltpu.VMEM_SHARED((NPAD,), jnp.float32),
            pltpu.SemaphoreType.DMA((2,)),
        ],
    )


def _sc_segment_mean(xs_flat, src_g, dst_g, ones_h, zeros_h):
    return _sc_segment_mean_fn()(xs_flat, src_g, dst_g, ones_h, zeros_h)


def _expand_body(m0_ref, m1_ref, x0_ref, x1_ref, wl_ref, bl_ref, wr_ref,
                 o0_ref, o1_ref):
    o0_ref[...] = m0_ref[...] * wl_ref[0] + bl_ref[0] + x0_ref[...] * wr_ref[0]
    o1_ref[...] = m1_ref[...] * wl_ref[1] + bl_ref[1] + x1_ref[...] * wr_ref[1]


def _expand_call(mean0, mean1, xs0, xs1, wl_all, bl_all, wr_all,
                 interpret=False):
    be = 512
    vec = pl.BlockSpec((be, 1), lambda i: (i, 0))
    wrow = pl.BlockSpec((2, 1, D), lambda i: (0, 0, 0))
    return pl.pallas_call(
        _expand_body,
        grid=(NPAD // be,),
        in_specs=[vec, vec, vec, vec, wrow, wrow, wrow],
        out_specs=[
            pl.BlockSpec((be, D), lambda i: (i, 0)),
            pl.BlockSpec((be, D), lambda i: (i, 0)),
        ],
        out_shape=[
            jax.ShapeDtypeStruct((N, D), jnp.float32),
            jax.ShapeDtypeStruct((N, D), jnp.float32),
        ],
        interpret=interpret,
    )(mean0, mean1, xs0, xs1, wl_all, bl_all, wr_all)


def _prep_edges(edge_index_iu, edge_index_ui):
    pad_n = EPT - E
    ar = jnp.arange(pad_n, dtype=jnp.int32)
    pad_src = ar % N
    pad_dst = N + (ar % (NPAD - N))
    src0 = jnp.concatenate([edge_index_iu[0], pad_src])
    dst0 = jnp.concatenate([edge_index_iu[1], pad_dst])
    src1 = jnp.concatenate([edge_index_ui[0], pad_src])
    dst1 = jnp.concatenate([edge_index_ui[1], pad_dst])
    src_g = jnp.stack([src0, src1]).reshape(2, NS, CH, CHUNK)
    dst_g = jnp.stack([dst0, dst1]).reshape(2, NS, CH, CHUNK)
    return src_g, dst_g


def kernel(x_user, x_item, edge_index_ui, edge_index_iu,
           Wi_user, bi_user, Wo_user, bo_user,
           Wi_item, bi_item, Wo_item, bo_item,
           Wl_ui, bl_ui, Wr_ui, Wl_iu, bl_iu, Wr_iu):
    wi_all = jnp.stack([Wi_user, Wi_item])
    bi_all = jnp.stack([bi_user, bi_item]).reshape(2, 1, 3 * D)
    wo0_all = jnp.stack([Wo_user[0:1, :], Wo_item[0:1, :]])
    bo0_all = jnp.stack([bo_user[0:1], bo_item[0:1]]).reshape(2, 1, 1)

    xs_u, xs_i = _attn_call(x_user, x_item, wi_all, bi_all, wo0_all, bo0_all)

    src_g, dst_g = _prep_edges(edge_index_iu, edge_index_ui)
    pad1 = jnp.zeros((NPAD - N,), jnp.float32)
    xs_flat = jnp.concatenate(
        [xs_u[:, 0], pad1, xs_i[:, 0], pad1])
    ones_h = jnp.ones((1, CHUNK), jnp.float32)
    zeros_h = jnp.zeros((NPAD,), jnp.float32)
    mean_all = _sc_segment_mean(xs_flat, src_g, dst_g, ones_h, zeros_h)

    wl_all = jnp.stack([Wl_iu[:, 0], Wl_ui[:, 0]]).reshape(2, 1, D)
    bl_all = jnp.stack([bl_iu, bl_ui]).reshape(2, 1, D)
    wr_all = jnp.stack([Wr_iu[:, 0], Wr_ui[:, 0]]).reshape(2, 1, D)
    mean2 = mean_all.reshape(2, NPAD, 1)
    o0, o1 = _expand_call(mean2[0], mean2[1], xs_u, xs_i,
                          wl_all, bl_all, wr_all)
    return (o0, o1)
